# Initial kernel scaffold; baseline (speedup 1.0000x reference)
import jax
import jax.numpy as jnp
from jax.experimental import pallas as pl


def kernel(x, edge_index, W1, a_src1, a_dst1, b1, W2, a_src2, a_dst2, b2, Wout, bout):
    raise NotImplementedError("write your pallas kernel here")


# speedup vs baseline: 116.7783x; 116.7783x over previous
import dataclasses
import functools

import jax
import jax.numpy as jnp
from jax import lax
from jax.experimental import pallas as pl
from jax.experimental.pallas import tpu as pltpu
from jax.experimental.pallas import tpu_sc as plsc

N_NODES = 100000
N_EDGES = 6400000
NC = 2
NS = 16
NW = NC * NS
EPW = N_EDGES // NW
CH = 80
NCHUNK = EPW // CH
NQ = N_NODES // 4

_mesh = plsc.VectorSubcoreMesh(
    core_axis_name="c", subcore_axis_name="s", num_cores=NC, num_subcores=NS)

_cparams = pltpu.CompilerParams()
if "needs_layout_passes" in pltpu.CompilerParams.__dataclass_fields__:
    _cparams = dataclasses.replace(_cparams, needs_layout_passes=False)
if "use_tc_tiling_on_sc" in pltpu.CompilerParams.__dataclass_fields__:
    _cparams = dataclasses.replace(_cparams, use_tc_tiling_on_sc=False)


def _edge_kernel(layer1: bool):
    NP = N_NODES if layer1 else N_NODES // 2
    npsh = 0 if layer1 else 1

    @functools.partial(
        pl.kernel,
        out_type=[jax.ShapeDtypeStruct((NP, 16), jnp.float32),
                  jax.ShapeDtypeStruct((NP, 16), jnp.float32),
                  jax.ShapeDtypeStruct((NQ, 16), jnp.float32),
                  jax.ShapeDtypeStruct((NQ, 16), jnp.float32)],
        mesh=_mesh,
        compiler_params=_cparams,
        scratch_types=[
            pltpu.VMEM((CH,), jnp.int32),
            pltpu.VMEM((CH,), jnp.int32),
            pltpu.VMEM((CH,), jnp.int32),
            pltpu.VMEM((CH,), jnp.int32),
            pltpu.VMEM((CH,), jnp.int32),
            pltpu.VMEM((CH, 16), jnp.float32),
            pltpu.VMEM((CH, 16), jnp.float32),
            pltpu.VMEM((CH, 16), jnp.float32),
            pltpu.VMEM((CH, 16), jnp.float32),
            pltpu.VMEM((16,), jnp.float32),
            pltpu.VMEM_SHARED((NP, 16), jnp.float32),
            pltpu.VMEM_SHARED((NQ, 16), jnp.float32),
            pltpu.SemaphoreType.DMA,
            pltpu.SemaphoreType.DMA,
        ],
    )
    def edge_kernel(src_hbm, dst_hbm, spk_hbm, ad_hbm,
                    outp0, outp1, outd0, outd1,
                    sidx, didx, dpix, dqix, ridx, spkv, adv, rowsP, rowsD,
                    exb, accP, accD, sem0, sem1):
        cid = lax.axis_index("c")
        sid = lax.axis_index("s")
        wid = cid * NS + sid
        lane = lax.iota(jnp.int32, 16)
        e4 = lane >> 2
        h4 = lane & 3
        z16 = jnp.zeros((16,), jnp.float32)

        def fill_ridx(base):
            for t in range(CH // 16):
                ridx[pl.ds(16 * t, 16)] = lane + (base + 16 * t)

        def sweep(acc, ntot, body):
            per = -(-ntot // NS)
            nch = -(-per // CH)
            rbeg = jnp.minimum(sid * per, ntot - per)

            @pl.loop(0, nch)
            def _(k):
                fill_ridx(rbeg + jnp.minimum(k * CH, per - CH))
                body(acc)

        def zero_rows(buf):
            @pl.loop(0, CH)
            def _(j):
                buf[j, :] = z16

        zero_rows(rowsP)

        def zinit(acc):
            pltpu.sync_copy(rowsP, acc.at[ridx])

        sweep(accP, NP, zinit)
        sweep(accD, NQ, zinit)
        plsc.subcore_barrier()

        @pl.loop(0, NCHUNK)
        def _chunk(ci):
            base = pl.multiple_of(wid * EPW + ci * CH, 8)
            cp0 = pltpu.async_copy(src_hbm.at[pl.ds(base, CH)], sidx, sem0)
            cp1 = pltpu.async_copy(dst_hbm.at[pl.ds(base, CH)], didx, sem1)
            cp0.wait()
            cp1.wait()
            cp2 = pltpu.async_copy(spk_hbm.at[sidx], spkv, sem0)
            cp3 = pltpu.async_copy(ad_hbm.at[didx], adv, sem1)

            @pl.loop(0, CH, step=16)
            def _ix(o):
                dv = didx[pl.ds(o, 16)]
                dqix[pl.ds(o, 16)] = dv >> 2
                if npsh:
                    dpix[pl.ds(o, 16)] = dv >> npsh

            cp2.wait()
            cp3.wait()

            zero_rows(rowsD)
            if not layer1:
                zero_rows(rowsP)

            @pl.loop(0, CH, step=4)
            def _grp(j):
                asv = plsc.load_gather(spkv, [j + e4, h4])
                advv = plsc.load_gather(adv, [j + e4, h4])
                t = asv + advv
                ex = jnp.exp(jnp.maximum(t, 0.2 * t))
                exb[...] = ex
                dstv = plsc.load_gather(didx, [j + e4])
                plsc.store_scatter(rowsD, [j + e4, (dstv & 3) * 4 + h4], ex)
                if layer1:
                    for e in range(4):
                        exv = plsc.load_gather(exb, [e * 4 + e4])
                        row = jnp.zeros((16,), jnp.int32) + (j + e)
                        xv = plsc.load_gather(spkv, [row, h4 + 4])
                        rowsP[j + e, :] = exv * xv
                else:
                    c8 = lane & 7
                    for hf in range(2):
                        er = (lane >> 3) + 2 * hf
                        exv = plsc.load_gather(exb, [er * 4 + (c8 >> 1)])
                        pv = plsc.load_gather(spkv, [j + er, c8 + 4])
                        d2 = plsc.load_gather(didx, [j + er])
                        plsc.store_scatter(
                            rowsP, [j + er, (d2 & 1) * 8 + c8], exv * pv)

            pidx = didx if layer1 else dpix
            pltpu.sync_copy(rowsP, accP.at[pidx], add=True)
            pltpu.sync_copy(rowsD, accD.at[dqix], add=True)

        plsc.subcore_barrier()

        def drain(dst_ref):
            def body(acc):
                pltpu.sync_copy(acc.at[ridx], rowsP)
                pltpu.sync_copy(rowsP, dst_ref.at[ridx])
            return body

        @pl.when(cid == 0)
        def _():
            sweep(accP, NP, drain(outp0))
            sweep(accD, NQ, drain(outd0))

        @pl.when(cid == 1)
        def _():
            sweep(accP, NP, drain(outp1))
            sweep(accD, NQ, drain(outd1))

    return edge_kernel


_edge_l1 = _edge_kernel(True)
_edge_l2 = _edge_kernel(False)


def _leaky_exp(t):
    return jnp.exp(jnp.maximum(t, 0.2 * t))


def kernel(x, edge_index, W1, a_src1, a_dst1, b1, W2, a_src2, a_dst2, b2,
           Wout, bout):
    N = x.shape[0]
    f32 = jnp.float32
    src = edge_index[0]
    dst = edge_index[1]
    pad4 = jnp.zeros((N, 4), f32)
    pad8 = jnp.zeros((N, 8), f32)
    pad12 = jnp.zeros((N, 12), f32)

    W1h = W1.reshape(4, 4, 16)
    As1 = jnp.einsum("khc,hc->kh", W1h, a_src1)
    Ad1 = jnp.einsum("khc,hc->kh", W1h, a_dst1)
    as1 = x @ As1
    ad1 = x @ Ad1
    spk1 = jnp.concatenate([as1, x, pad8], axis=1)
    adp1 = jnp.concatenate([ad1, pad12], axis=1)
    p0, p1, d0, d1 = _edge_l1(src, dst, spk1, adp1)
    exs1 = _leaky_exp(as1 + ad1)
    s1 = (p0 + p1).reshape(N, 4, 4) + exs1[:, :, None] * x[:, None, :]
    denom1 = (d0 + d1).reshape(N, 4) + exs1 + 1e-16
    out1 = jnp.einsum("nhk,khc->nhc", s1, W1h) / denom1[:, :, None]
    h1 = jax.nn.elu(out1.reshape(N, 64) + b1)

    h2p = h1 @ W2
    h2r = h2p.reshape(N, 4, 2)
    as2 = (h2r * a_src2[None]).sum(-1)
    ad2 = (h2r * a_dst2[None]).sum(-1)
    spk2 = jnp.concatenate([as2, h2p, pad4], axis=1)
    adp2 = jnp.concatenate([ad2, pad12], axis=1)
    q0, q1, e0, e1 = _edge_l2(src, dst, spk2, adp2)
    exs2 = _leaky_exp(as2 + ad2)
    num2 = (q0 + q1).reshape(N, 4, 2) + exs2[:, :, None] * h2r
    denom2 = (e0 + e1).reshape(N, 4) + exs2 + 1e-16
    h2 = jax.nn.elu((num2 / denom2[:, :, None]).reshape(N, 8) + b2)

    pooled = jnp.mean(h2, axis=0, keepdims=True)
    return jax.nn.sigmoid(pooled @ Wout + bout)


# speedup vs baseline: 119.1297x; 1.0201x over previous
import dataclasses
import functools

import jax
import jax.numpy as jnp
from jax import lax
from jax.experimental import pallas as pl
from jax.experimental.pallas import tpu as pltpu
from jax.experimental.pallas import tpu_sc as plsc

N_NODES = 100000
N_EDGES = 6400000
NC = 2
NS = 16
NW = NC * NS
EPW = N_EDGES // NW
CH = 80
NCHUNK = EPW // CH
NQ = N_NODES // 4

_mesh = plsc.VectorSubcoreMesh(
    core_axis_name="c", subcore_axis_name="s", num_cores=NC, num_subcores=NS)

_cparams = pltpu.CompilerParams()
if "needs_layout_passes" in pltpu.CompilerParams.__dataclass_fields__:
    _cparams = dataclasses.replace(_cparams, needs_layout_passes=False)
if "use_tc_tiling_on_sc" in pltpu.CompilerParams.__dataclass_fields__:
    _cparams = dataclasses.replace(_cparams, use_tc_tiling_on_sc=False)


def _edge_kernel(layer1: bool):
    NP = N_NODES if layer1 else N_NODES // 2
    npsh = 0 if layer1 else 1

    @functools.partial(
        pl.kernel,
        out_type=[jax.ShapeDtypeStruct((NP, 16), jnp.float32),
                  jax.ShapeDtypeStruct((NP, 16), jnp.float32),
                  jax.ShapeDtypeStruct((NQ, 16), jnp.float32),
                  jax.ShapeDtypeStruct((NQ, 16), jnp.float32)],
        mesh=_mesh,
        compiler_params=_cparams,
        scratch_types=[
            pltpu.VMEM((CH,), jnp.int32),
            pltpu.VMEM((CH,), jnp.int32),
            pltpu.VMEM((CH,), jnp.int32),
            pltpu.VMEM((CH,), jnp.int32),
            pltpu.VMEM((CH,), jnp.int32),
            pltpu.VMEM((CH, 16), jnp.float32),
            pltpu.VMEM((CH, 16), jnp.float32),
            pltpu.VMEM((CH, 16), jnp.float32),
            pltpu.VMEM((CH, 16), jnp.float32),
            pltpu.VMEM((16,), jnp.float32),
            pltpu.VMEM_SHARED((NP, 16), jnp.float32),
            pltpu.VMEM_SHARED((NQ, 16), jnp.float32),
            pltpu.SemaphoreType.DMA,
            pltpu.SemaphoreType.DMA,
        ],
    )
    def edge_kernel(src_hbm, dst_hbm, spk_hbm, ad_hbm,
                    outp0, outp1, outd0, outd1,
                    sidx, didx, dpix, dqix, ridx, spkv, adv, rowsP, rowsD,
                    exb, accP, accD, sem0, sem1):
        cid = lax.axis_index("c")
        sid = lax.axis_index("s")
        wid = cid * NS + sid
        lane = lax.iota(jnp.int32, 16)
        e4 = lane >> 2
        h4 = lane & 3
        z16 = jnp.zeros((16,), jnp.float32)

        def fill_ridx(base):
            for t in range(CH // 16):
                ridx[pl.ds(16 * t, 16)] = lane + (base + 16 * t)

        def sweep(acc, ntot, body):
            per = -(-ntot // NS)
            nch = -(-per // CH)
            rbeg = jnp.minimum(sid * per, ntot - per)

            @pl.loop(0, nch)
            def _(k):
                fill_ridx(rbeg + jnp.minimum(k * CH, per - CH))
                body(acc)

        def zero_rows(buf):
            @pl.loop(0, CH)
            def _(j):
                buf[j, :] = z16

        zero_rows(rowsP)

        def zinit(acc):
            pltpu.sync_copy(rowsP, acc.at[ridx])

        sweep(accP, NP, zinit)
        sweep(accD, NQ, zinit)
        plsc.subcore_barrier()

        @pl.loop(0, NCHUNK)
        def _chunk(ci):
            base = pl.multiple_of(wid * EPW + ci * CH, 8)
            cp0 = pltpu.async_copy(src_hbm.at[pl.ds(base, CH)], sidx, sem0)
            cp1 = pltpu.async_copy(dst_hbm.at[pl.ds(base, CH)], didx, sem1)
            cp0.wait()
            cp1.wait()
            cp2 = pltpu.async_copy(spk_hbm.at[sidx], spkv, sem0)
            cp3 = pltpu.async_copy(ad_hbm.at[didx], adv, sem1)

            @pl.loop(0, CH, step=16)
            def _ix(o):
                dv = didx[pl.ds(o, 16)]
                dqix[pl.ds(o, 16)] = dv >> 2
                if npsh:
                    dpix[pl.ds(o, 16)] = dv >> npsh

            cp2.wait()
            cp3.wait()

            zero_rows(rowsD)
            if not layer1:
                zero_rows(rowsP)

            @pl.loop(0, CH, step=4)
            def _grp(j):
                asv = plsc.load_gather(spkv, [j + e4, h4])
                advv = plsc.load_gather(adv, [j + e4, h4])
                t = asv + advv
                ex = jnp.exp(jnp.maximum(t, 0.2 * t))
                exb[...] = ex
                dstv = plsc.load_gather(didx, [j + e4])
                plsc.store_scatter(rowsD, [j + e4, (dstv & 3) * 4 + h4], ex)
                if layer1:
                    for e in range(4):
                        exv = plsc.load_gather(exb, [e * 4 + e4])
                        row = jnp.zeros((16,), jnp.int32) + (j + e)
                        xv = plsc.load_gather(spkv, [row, h4 + 4])
                        rowsP[j + e, :] = exv * xv
                else:
                    c8 = lane & 7
                    for hf in range(2):
                        er = (lane >> 3) + 2 * hf
                        exv = plsc.load_gather(exb, [er * 4 + (c8 >> 1)])
                        pv = plsc.load_gather(spkv, [j + er, c8 + 4])
                        d2 = plsc.load_gather(didx, [j + er])
                        plsc.store_scatter(
                            rowsP, [j + er, (d2 & 1) * 8 + c8], exv * pv)

            pidx = didx if layer1 else dpix
            sc0 = pltpu.async_copy(rowsP, accP.at[pidx], sem0, add=True)
            sc1 = pltpu.async_copy(rowsD, accD.at[dqix], sem1, add=True)
            sc0.wait()
            sc1.wait()

        plsc.subcore_barrier()

        def drain(dst_ref):
            def body(acc):
                pltpu.sync_copy(acc.at[ridx], rowsP)
                pltpu.sync_copy(rowsP, dst_ref.at[ridx])
            return body

        @pl.when(cid == 0)
        def _():
            sweep(accP, NP, drain(outp0))
            sweep(accD, NQ, drain(outd0))

        @pl.when(cid == 1)
        def _():
            sweep(accP, NP, drain(outp1))
            sweep(accD, NQ, drain(outd1))

    return edge_kernel


_edge_l1 = _edge_kernel(True)
_edge_l2 = _edge_kernel(False)


def _leaky_exp(t):
    return jnp.exp(jnp.maximum(t, 0.2 * t))


def kernel(x, edge_index, W1, a_src1, a_dst1, b1, W2, a_src2, a_dst2, b2,
           Wout, bout):
    N = x.shape[0]
    f32 = jnp.float32
    src = edge_index[0]
    dst = edge_index[1]
    pad4 = jnp.zeros((N, 4), f32)
    pad8 = jnp.zeros((N, 8), f32)
    pad12 = jnp.zeros((N, 12), f32)

    W1h = W1.reshape(4, 4, 16)
    As1 = jnp.einsum("khc,hc->kh", W1h, a_src1)
    Ad1 = jnp.einsum("khc,hc->kh", W1h, a_dst1)
    as1 = x @ As1
    ad1 = x @ Ad1
    spk1 = jnp.concatenate([as1, x, pad8], axis=1)
    adp1 = jnp.concatenate([ad1, pad12], axis=1)
    p0, p1, d0, d1 = _edge_l1(src, dst, spk1, adp1)
    exs1 = _leaky_exp(as1 + ad1)
    s1 = (p0 + p1).reshape(N, 4, 4) + exs1[:, :, None] * x[:, None, :]
    denom1 = (d0 + d1).reshape(N, 4) + exs1 + 1e-16
    out1 = jnp.einsum("nhk,khc->nhc", s1, W1h) / denom1[:, :, None]
    h1 = jax.nn.elu(out1.reshape(N, 64) + b1)

    h2p = h1 @ W2
    h2r = h2p.reshape(N, 4, 2)
    as2 = (h2r * a_src2[None]).sum(-1)
    ad2 = (h2r * a_dst2[None]).sum(-1)
    spk2 = jnp.concatenate([as2, h2p, pad4], axis=1)
    adp2 = jnp.concatenate([ad2, pad12], axis=1)
    q0, q1, e0, e1 = _edge_l2(src, dst, spk2, adp2)
    exs2 = _leaky_exp(as2 + ad2)
    num2 = (q0 + q1).reshape(N, 4, 2) + exs2[:, :, None] * h2r
    denom2 = (e0 + e1).reshape(N, 4) + exs2 + 1e-16
    h2 = jax.nn.elu((num2 / denom2[:, :, None]).reshape(N, 8) + b2)

    pooled = jnp.mean(h2, axis=0, keepdims=True)
    return jax.nn.sigmoid(pooled @ Wout + bout)
